# Initial kernel scaffold; baseline (speedup 1.0000x reference)
import jax
import jax.numpy as jnp
from jax.experimental import pallas as pl


def kernel(h, e, spatial_pos, edge_index, Wq, bq, Wk, bk, Wv, bv, We, be, Wp, bp, ln_g, ln_b, pos_embedding, temperature):
    raise NotImplementedError("write your pallas kernel here")


# speedup vs baseline: 55.0719x; 55.0719x over previous
import functools

import jax
import jax.numpy as jnp
from jax import lax
from jax.experimental import pallas as pl
from jax.experimental.pallas import tpu as pltpu
from jax.experimental.pallas import tpu_sc as plsc

N = 10000
E = 320000
H = 8
D = 16
OUT = H * D

_BN = 2000
_BE = 2000

_NS = 16
_B = 128
_NBLK = E // _B
_BASE_BLK = _NBLK // _NS
_EXTRA = _NBLK - _BASE_BLK * _NS
_RPT = N // _NS
_RCH = 125


def _ln(y, g, b):
    mu = jnp.mean(y, axis=-1, keepdims=True)
    yc = y - mu
    var = jnp.mean(yc * yc, axis=-1, keepdims=True)
    return yc * lax.rsqrt(var + 1e-5) * g + b


def _qkv_body(temp_ref, h_ref, wq_ref, bq_ref, wk_ref, bk_ref, wv_ref, bv_ref,
              g_ref, b_ref, q_out, k_out, v_out):
    x = h_ref[...]
    g = g_ref[...]
    b = b_ref[...]
    inv = 1.0 / (4.0 * temp_ref[0])
    q = _ln(jnp.dot(x, wq_ref[...], preferred_element_type=jnp.float32)
            + bq_ref[...], g, b) * inv
    k = _ln(jnp.dot(x, wk_ref[...], preferred_element_type=jnp.float32)
            + bk_ref[...], g, b)
    v = _ln(jnp.dot(x, wv_ref[...], preferred_element_type=jnp.float32)
            + bv_ref[...], g, b)
    q_out[0] = q[:, :64]
    q_out[1] = q[:, 64:]
    k_out[0] = k[:, :64]
    k_out[1] = k[:, 64:]
    v_out[0] = v[:, :64]
    v_out[1] = v[:, 64:]


def _qkv_call(temperature, h, Wq, bq, Wk, bk, Wv, bv, g, b):
    wspec = pl.BlockSpec((OUT, OUT), lambda i: (0, 0))
    bspec = pl.BlockSpec((OUT,), lambda i: (0,))
    return pl.pallas_call(
        _qkv_body,
        grid=(N // _BN,),
        in_specs=[
            pl.BlockSpec(memory_space=pltpu.SMEM),
            pl.BlockSpec((_BN, OUT), lambda i: (i, 0)),
            wspec, bspec, wspec, bspec, wspec, bspec, bspec, bspec,
        ],
        out_specs=[pl.BlockSpec((2, _BN, 64), lambda i: (0, i, 0))] * 3,
        out_shape=[jax.ShapeDtypeStruct((2, N, 64), jnp.float32)] * 3,
    )(temperature, h, Wq, bq, Wk, bk, Wv, bv, g, b)


def _base_body(e_ref, sp_ref, we_ref, be_ref, wp_ref, bp_ref, pos_ref,
               g_ref, b_ref, out_ref):
    g = g_ref[...]
    b = b_ref[...]
    pe = _ln(jnp.dot(e_ref[...], we_ref[...], preferred_element_type=jnp.float32)
             + be_ref[...], g, b)
    lp = _ln(jnp.dot(sp_ref[...], wp_ref[...], preferred_element_type=jnp.float32)
             + bp_ref[...] + pos_ref[...], g, b)
    out_ref[...] = pe + lp


def _base_call(e, sp, We, be, Wp, bp, pos, g, b):
    wspec = pl.BlockSpec((OUT, OUT), lambda i: (0, 0))
    bspec = pl.BlockSpec((OUT,), lambda i: (0,))
    return pl.pallas_call(
        _base_body,
        grid=(E // _BE,),
        in_specs=[
            pl.BlockSpec((_BE, OUT), lambda i: (i, 0)),
            pl.BlockSpec((_BE, OUT), lambda i: (i, 0)),
            wspec, bspec, wspec, bspec,
            pl.BlockSpec((1, OUT), lambda i: (0, 0)),
            bspec, bspec,
        ],
        out_specs=pl.BlockSpec((_BE, OUT), lambda i: (i, 0)),
        out_shape=jax.ShapeDtypeStruct((E, OUT), jnp.float32),
    )(e, sp, We, be, Wp, bp, pos, g, b)


_MESH = plsc.VectorSubcoreMesh(core_axis_name="c", subcore_axis_name="s")


@functools.partial(
    pl.kernel,
    out_type=[
        jax.ShapeDtypeStruct((N, OUT), jnp.float32),
        jax.ShapeDtypeStruct((E, OUT), jnp.float32),
    ],
    mesh=_MESH,
    compiler_params=pltpu.CompilerParams(use_tc_tiling_on_sc=False,
                                          needs_layout_passes=False),
    scratch_types=[
        pltpu.VMEM((_B,), jnp.int32),
        pltpu.VMEM((_B,), jnp.int32),
        pltpu.VMEM((_B,), jnp.int32),
        pltpu.VMEM((_B,), jnp.int32),
        pltpu.VMEM((_B, 64), jnp.float32),
        pltpu.VMEM((_B, 64), jnp.float32),
        pltpu.VMEM((_B, 64), jnp.float32),
        pltpu.VMEM((_B, 64), jnp.float32),
        pltpu.VMEM_SHARED((N, 64), jnp.float32),
        pltpu.VMEM_SHARED((N, 64), jnp.float32),
        pltpu.SemaphoreType.DMA,
        pltpu.SemaphoreType.DMA,
        pltpu.SemaphoreType.DMA,
        pltpu.SemaphoreType.DMA,
    ],
)
def _sc_attn(q_hbm, k_hbm, v_hbm, base_hbm, src_hbm, dst_hbm,
             hout_hbm, eout_hbm,
             sraw, draw, gsrc, gdst, kbuf, qbuf, vbuf, bbuf,
             wv_sh, z_sh, sem0, sem1, sem2, sem3):
    c = lax.axis_index("c")
    s = lax.axis_index("s")
    cN = c * N
    col0 = c * 64

    zv = jnp.zeros((16,), jnp.float32)

    def _zero_bbuf(i, _):
        for j in range(4):
            bbuf[i, pl.ds(j * 16, 16)] = zv
        return 0

    lax.fori_loop(0, _B, _zero_bbuf, 0)
    for t in range(_RPT // _RCH):
        r0 = s * _RPT + t * _RCH
        pltpu.sync_copy(bbuf.at[pl.ds(0, _RCH)], wv_sh.at[pl.ds(r0, _RCH)])
        pltpu.sync_copy(bbuf.at[pl.ds(0, _RCH)], z_sh.at[pl.ds(r0, _RCH)])
    plsc.subcore_barrier()

    nblk = _BASE_BLK + jnp.where(s < _EXTRA, 1, 0)

    def _edge_block(j, _):
        off = (s + j * _NS) * _B
        pltpu.sync_copy(src_hbm.at[pl.ds(off, _B)], sraw)
        pltpu.sync_copy(dst_hbm.at[pl.ds(off, _B)], draw)

        def _bias(i, _):
            o = i * 16
            gsrc[pl.ds(o, 16)] = sraw[pl.ds(o, 16)] + cN
            gdst[pl.ds(o, 16)] = draw[pl.ds(o, 16)] + cN
            return 0

        lax.fori_loop(0, _B // 16, _bias, 0)

        cpk = pltpu.async_copy(k_hbm.at[gsrc], kbuf, sem0)
        cpq = pltpu.async_copy(q_hbm.at[gdst], qbuf, sem1)
        cpv = pltpu.async_copy(v_hbm.at[gsrc], vbuf, sem2)
        cpb = pltpu.async_copy(
            base_hbm.at[pl.ds(off, _B), pl.ds(col0, 64)], bbuf, sem3)
        cpk.wait()
        cpq.wait()
        cpv.wait()
        cpb.wait()

        def _edge(i, _):
            for hh in range(4):
                sl = pl.ds(hh * 16, 16)
                sc = kbuf[i, sl] * qbuf[i, sl] + bbuf[i, sl]
                sc = jnp.minimum(jnp.maximum(sc, -5.0), 5.0)
                p = jnp.exp(sc)
                r = p / jnp.sum(p)
                bbuf[i, sl] = r
                vbuf[i, sl] = vbuf[i, sl] * r
            return 0

        lax.fori_loop(0, _B, _edge, 0)

        pltpu.sync_copy(bbuf, eout_hbm.at[pl.ds(off, _B), pl.ds(col0, 64)])
        pltpu.sync_copy(bbuf, z_sh.at[draw], add=True)
        pltpu.sync_copy(vbuf, wv_sh.at[draw], add=True)
        return 0

    lax.fori_loop(0, nblk, _edge_block, 0)
    plsc.subcore_barrier()

    for t in range(_RPT // _RCH):
        r0 = s * _RPT + t * _RCH
        pltpu.sync_copy(wv_sh.at[pl.ds(r0, _RCH)], kbuf.at[pl.ds(0, _RCH)])
        pltpu.sync_copy(z_sh.at[pl.ds(r0, _RCH)], qbuf.at[pl.ds(0, _RCH)])

        def _div(i, _):
            for hh in range(4):
                sl = pl.ds(hh * 16, 16)
                kbuf[i, sl] = kbuf[i, sl] / (qbuf[i, sl] + 1e-6)
            return 0

        lax.fori_loop(0, _RCH, _div, 0)
        pltpu.sync_copy(kbuf.at[pl.ds(0, _RCH)],
                        hout_hbm.at[pl.ds(r0, _RCH), pl.ds(col0, 64)])


def kernel(h, e, spatial_pos, edge_index, Wq, bq, Wk, bk, Wv, bv, We, be,
           Wp, bp, ln_g, ln_b, pos_embedding, temperature):
    qs, ks, vs = _qkv_call(temperature, h, Wq, bq, Wk, bk, Wv, bv, ln_g, ln_b)
    base = _base_call(e, spatial_pos, We, be, Wp, bp, pos_embedding, ln_g, ln_b)
    src = edge_index[0]
    dst = edge_index[1]
    h_out, e_out = _sc_attn(
        qs.reshape(2 * N, 64), ks.reshape(2 * N, 64), vs.reshape(2 * N, 64),
        base, src, dst)
    return h_out.reshape(N, H, D), e_out.reshape(E, H, D)


# speedup vs baseline: 69.8083x; 1.2676x over previous
import functools

import jax
import jax.numpy as jnp
from jax import lax
from jax.experimental import pallas as pl
from jax.experimental.pallas import tpu as pltpu
from jax.experimental.pallas import tpu_sc as plsc

N = 10000
E = 320000
H = 8
D = 16
OUT = H * D

_BN = 2000
_BE = 2000

_NS = 16
_B = 80
_EPT = E // _NS
_FULL = _EPT // _B
_NPAIR = _FULL // 2
_RPT = N // _NS
_RCHUNKS = [(i * _B, _B) for i in range(_RPT // _B)] + [
    (_RPT - _RPT % _B, _RPT % _B)]


def _ln(y, g, b):
    mu = jnp.mean(y, axis=-1, keepdims=True)
    yc = y - mu
    var = jnp.mean(yc * yc, axis=-1, keepdims=True)
    return yc * lax.rsqrt(var + 1e-5) * g + b


def _qkv_body(temp_ref, h_ref, wq_ref, bq_ref, wk_ref, bk_ref, wv_ref, bv_ref,
              g_ref, b_ref, q_out, k_out, v_out):
    x = h_ref[...]
    g = g_ref[...]
    b = b_ref[...]
    inv = 1.0 / (4.0 * temp_ref[0])
    q = _ln(jnp.dot(x, wq_ref[...], preferred_element_type=jnp.float32)
            + bq_ref[...], g, b) * inv
    k = _ln(jnp.dot(x, wk_ref[...], preferred_element_type=jnp.float32)
            + bk_ref[...], g, b)
    v = _ln(jnp.dot(x, wv_ref[...], preferred_element_type=jnp.float32)
            + bv_ref[...], g, b)
    q_out[0] = q[:, :64]
    q_out[1] = q[:, 64:]
    k_out[0] = k[:, :64]
    k_out[1] = k[:, 64:]
    v_out[0] = v[:, :64]
    v_out[1] = v[:, 64:]


def _qkv_call(temperature, h, Wq, bq, Wk, bk, Wv, bv, g, b):
    wspec = pl.BlockSpec((OUT, OUT), lambda i: (0, 0))
    bspec = pl.BlockSpec((OUT,), lambda i: (0,))
    return pl.pallas_call(
        _qkv_body,
        grid=(N // _BN,),
        in_specs=[
            pl.BlockSpec(memory_space=pltpu.SMEM),
            pl.BlockSpec((_BN, OUT), lambda i: (i, 0)),
            wspec, bspec, wspec, bspec, wspec, bspec, bspec, bspec,
        ],
        out_specs=[pl.BlockSpec((2, _BN, 64), lambda i: (0, i, 0))] * 3,
        out_shape=[jax.ShapeDtypeStruct((2, N, 64), jnp.float32)] * 3,
    )(temperature, h, Wq, bq, Wk, bk, Wv, bv, g, b)


def _base_body(e_ref, sp_ref, we_ref, be_ref, wp_ref, bp_ref, pos_ref,
               g_ref, b_ref, out_ref):
    g = g_ref[...]
    b = b_ref[...]
    pe = _ln(jnp.dot(e_ref[...], we_ref[...], preferred_element_type=jnp.float32)
             + be_ref[...], g, b)
    lp = _ln(jnp.dot(sp_ref[...], wp_ref[...], preferred_element_type=jnp.float32)
             + bp_ref[...] + pos_ref[...], g, b)
    out_ref[...] = pe + lp


def _base_call(e, sp, We, be, Wp, bp, pos, g, b):
    wspec = pl.BlockSpec((OUT, OUT), lambda i: (0, 0))
    bspec = pl.BlockSpec((OUT,), lambda i: (0,))
    return pl.pallas_call(
        _base_body,
        grid=(E // _BE,),
        in_specs=[
            pl.BlockSpec((_BE, OUT), lambda i: (i, 0)),
            pl.BlockSpec((_BE, OUT), lambda i: (i, 0)),
            wspec, bspec, wspec, bspec,
            pl.BlockSpec((1, OUT), lambda i: (0, 0)),
            bspec, bspec,
        ],
        out_specs=pl.BlockSpec((_BE, OUT), lambda i: (i, 0)),
        out_shape=jax.ShapeDtypeStruct((E, OUT), jnp.float32),
    )(e, sp, We, be, Wp, bp, pos, g, b)


_MESH = plsc.VectorSubcoreMesh(core_axis_name="c", subcore_axis_name="s")


@functools.partial(
    pl.kernel,
    out_type=[
        jax.ShapeDtypeStruct((N, OUT), jnp.float32),
        jax.ShapeDtypeStruct((E, OUT), jnp.float32),
    ],
    mesh=_MESH,
    compiler_params=pltpu.CompilerParams(use_tc_tiling_on_sc=False,
                                          needs_layout_passes=False),
    scratch_types=[
        pltpu.VMEM((_B,), jnp.int32),
        pltpu.VMEM((_B,), jnp.int32),
        pltpu.VMEM((_B,), jnp.int32),
        pltpu.VMEM((_B,), jnp.int32),
        pltpu.VMEM((_B, 64), jnp.float32),
        pltpu.VMEM((_B, 64), jnp.float32),
        pltpu.VMEM((_B, 64), jnp.float32),
        pltpu.VMEM((_B, 64), jnp.float32),
        pltpu.VMEM((_B,), jnp.int32),
        pltpu.VMEM((_B,), jnp.int32),
        pltpu.VMEM((_B,), jnp.int32),
        pltpu.VMEM((_B,), jnp.int32),
        pltpu.VMEM((_B, 64), jnp.float32),
        pltpu.VMEM((_B, 64), jnp.float32),
        pltpu.VMEM((_B, 64), jnp.float32),
        pltpu.VMEM((_B, 64), jnp.float32),
        pltpu.VMEM_SHARED((N, 64), jnp.float32),
        pltpu.VMEM_SHARED((N, 64), jnp.float32),
        pltpu.SemaphoreType.DMA,
        pltpu.SemaphoreType.DMA,
        pltpu.SemaphoreType.DMA,
        pltpu.SemaphoreType.DMA,
    ],
)
def _sc_attn(q_hbm, k_hbm, v_hbm, base_hbm, src_hbm, dst_hbm,
             hout_hbm, eout_hbm,
             sraw0, draw0, gsrc0, gdst0, kbuf0, qbuf0, vbuf0, bbuf0,
             sraw1, draw1, gsrc1, gdst1, kbuf1, qbuf1, vbuf1, bbuf1,
             wv_sh, z_sh, semg0, semg1, semw0, semw1):
    c = lax.axis_index("c")
    s = lax.axis_index("s")
    cN = c * N
    col0 = c * 64
    ebase = s * _EPT

    sets = [
        (sraw0, draw0, gsrc0, gdst0, kbuf0, qbuf0, vbuf0, bbuf0, semg0, semw0),
        (sraw1, draw1, gsrc1, gdst1, kbuf1, qbuf1, vbuf1, bbuf1, semg1, semw1),
    ]

    def _prefetch(si, off, nb):
        sraw, draw, gsrc, gdst, kbuf, qbuf, vbuf, bbuf, semg, _ = sets[si]
        ci = pltpu.async_copy(src_hbm.at[pl.ds(off, nb)], sraw, semg)
        cj = pltpu.async_copy(dst_hbm.at[pl.ds(off, nb)], draw, semg)
        ci.wait()
        cj.wait()

        def _bias(i, _):
            o = i * 16
            gsrc[pl.ds(o, 16)] = sraw[pl.ds(o, 16)] + cN
            gdst[pl.ds(o, 16)] = draw[pl.ds(o, 16)] + cN
            return 0

        lax.fori_loop(0, nb // 16, _bias, 0)

        pltpu.async_copy(k_hbm.at[gsrc], kbuf, semg)
        pltpu.async_copy(q_hbm.at[gdst], qbuf, semg)
        pltpu.async_copy(v_hbm.at[gsrc], vbuf, semg)
        pltpu.async_copy(base_hbm.at[pl.ds(off, nb), pl.ds(col0, 64)],
                         bbuf, semg)

    def _wait_gathers(si, off, nb):
        sraw, draw, gsrc, gdst, kbuf, qbuf, vbuf, bbuf, semg, _ = sets[si]
        pltpu.make_async_copy(k_hbm.at[gsrc], kbuf, semg).wait()
        pltpu.make_async_copy(q_hbm.at[gdst], qbuf, semg).wait()
        pltpu.make_async_copy(v_hbm.at[gsrc], vbuf, semg).wait()
        pltpu.make_async_copy(base_hbm.at[pl.ds(off, nb), pl.ds(col0, 64)],
                              bbuf, semg).wait()

    def _compute(si, nb):
        _, _, _, _, kbuf, qbuf, vbuf, bbuf, _, _ = sets[si]

        def _edge(i, _):
            for hh in range(4):
                sl = pl.ds(hh * 16, 16)
                sc = kbuf[i, sl] * qbuf[i, sl] + bbuf[i, sl]
                sc = jnp.minimum(jnp.maximum(sc, -5.0), 5.0)
                p = jnp.exp(sc)
                r = p / jnp.sum(p)
                bbuf[i, sl] = r
                vbuf[i, sl] = vbuf[i, sl] * r
            return 0

        lax.fori_loop(0, nb, _edge, 0)

    def _fire_writes(si, off):
        _, draw, _, _, _, _, vbuf, bbuf, _, semw = sets[si]
        pltpu.async_copy(bbuf, eout_hbm.at[pl.ds(off, _B), pl.ds(col0, 64)],
                         semw)
        pltpu.sync_copy(bbuf, z_sh.at[draw], add=True)
        pltpu.sync_copy(vbuf, wv_sh.at[draw], add=True)

    def _drain_writes(si, off):
        _, _, _, _, _, _, _, bbuf, _, semw = sets[si]
        pltpu.make_async_copy(bbuf, eout_hbm.at[pl.ds(off, _B),
                                                pl.ds(col0, 64)], semw).wait()

    zv = jnp.zeros((16,), jnp.float32)

    def _zb(i, _):
        for j in range(4):
            bbuf0[i, pl.ds(j * 16, 16)] = zv
        return 0

    lax.fori_loop(0, _B, _zb, 0)

    for (ro, sz) in _RCHUNKS:
        r0 = s * _RPT + ro
        pltpu.sync_copy(bbuf0.at[pl.ds(0, sz)], wv_sh.at[pl.ds(r0, sz)])
        pltpu.sync_copy(bbuf0.at[pl.ds(0, sz)], z_sh.at[pl.ds(r0, sz)])
    plsc.subcore_barrier()

    _prefetch(0, ebase, _B)
    _prefetch(1, ebase + _B, _B)

    def _pair(j2, _):
        off0 = ebase + (2 * j2) * _B
        off1 = off0 + _B
        _wait_gathers(0, off0, _B)
        _compute(0, _B)
        _fire_writes(0, off0)
        _drain_writes(0, off0)
        _prefetch(0, off0 + 2 * _B, _B)

        _wait_gathers(1, off1, _B)
        _compute(1, _B)
        _fire_writes(1, off1)
        _drain_writes(1, off1)
        _prefetch(1, off1 + 2 * _B, _B)
        return 0

    lax.fori_loop(0, _NPAIR - 1, _pair, 0)
    last0 = ebase + (_FULL - 2) * _B
    for si, off in ((0, last0), (1, last0 + _B)):
        _wait_gathers(si, off, _B)
        _compute(si, _B)
        _fire_writes(si, off)
        _drain_writes(si, off)

    plsc.subcore_barrier()

    for (ro, sz) in _RCHUNKS:
        r0 = s * _RPT + ro
        pltpu.sync_copy(wv_sh.at[pl.ds(r0, sz)], kbuf0.at[pl.ds(0, sz)])
        pltpu.sync_copy(z_sh.at[pl.ds(r0, sz)], qbuf0.at[pl.ds(0, sz)])

        def _dv(i, _):
            for hh in range(4):
                sl = pl.ds(hh * 16, 16)
                kbuf0[i, sl] = kbuf0[i, sl] / (qbuf0[i, sl] + 1e-6)
            return 0

        lax.fori_loop(0, sz, _dv, 0)

        pltpu.sync_copy(kbuf0.at[pl.ds(0, sz)],
                        hout_hbm.at[pl.ds(r0, sz), pl.ds(col0, 64)])


def kernel(h, e, spatial_pos, edge_index, Wq, bq, Wk, bk, Wv, bv, We, be,
           Wp, bp, ln_g, ln_b, pos_embedding, temperature):
    qs, ks, vs = _qkv_call(temperature, h, Wq, bq, Wk, bk, Wv, bv, ln_g, ln_b)
    base = _base_call(e, spatial_pos, We, be, Wp, bp, pos_embedding, ln_g, ln_b)
    src = edge_index[0]
    dst = edge_index[1]
    h_out, e_out = _sc_attn(
        qs.reshape(2 * N, 64), ks.reshape(2 * N, 64), vs.reshape(2 * N, 64),
        base, src, dst)
    return h_out.reshape(N, H, D), e_out.reshape(E, H, D)
